# baseline (speedup 1.0000x reference)
import functools
import jax
import jax.numpy as jnp
from jax import lax
from jax.experimental import pallas as pl
from jax.experimental.pallas import tpu as pltpu
from jax.experimental.pallas import tpu_sc as plsc

T = 2048
D = 768
E = 8
FF = 768
BT = 256
NT = (2 * T) // BT + E
TOT = NT * BT
NW = 32
CH = T // NW


def _router_body(x_ref, gw_ref, slots_ref, wts_ref, te_ref):
    x = x_ref[...]
    logits = lax.dot_general(x, gw_ref[...], (((1,), (1,)), ((), ())),
                             preferred_element_type=jnp.float32)
    lane = lax.broadcasted_iota(jnp.int32, (T, E), 1)
    i1 = jnp.argmax(logits, axis=-1)[:, None]
    l1 = jnp.max(logits, axis=-1, keepdims=True)
    masked = jnp.where(lane == i1, -jnp.inf, logits)
    i2 = jnp.argmax(masked, axis=-1)[:, None]
    l2 = jnp.max(masked, axis=-1, keepdims=True)
    z = jnp.exp(l2 - l1)
    w1 = 1.0 / (1.0 + z)
    w2 = 1.0 - w1

    onehot = jnp.logical_or(lane == i1, lane == i2).astype(jnp.bfloat16)
    r_iota = lax.broadcasted_iota(jnp.int32, (T, T), 0)
    c_iota = lax.broadcasted_iota(jnp.int32, (T, T), 1)
    lt = (c_iota < r_iota).astype(jnp.bfloat16)
    ranks = lax.dot_general(lt, onehot, (((1,), (0,)), ((), ())),
                            preferred_element_type=jnp.float32)
    counts = jnp.sum(onehot.astype(jnp.float32), axis=0, keepdims=True)
    nt = jnp.floor((counts + (BT - 1)) / BT)
    ei = lax.broadcasted_iota(jnp.int32, (E, E), 0)
    ej = lax.broadcasted_iota(jnp.int32, (E, E), 1)
    m = (ei < ej).astype(jnp.float32)
    tb = lax.dot_general(nt, m, (((1,), (0,)), ((), ())),
                         preferred_element_type=jnp.float32)
    base = tb * BT

    slot_a = jnp.sum(jnp.where(lane == i1, ranks + base, 0.0), axis=-1,
                     keepdims=True)
    slot_b = jnp.sum(jnp.where(lane == i2, ranks + base, 0.0), axis=-1,
                     keepdims=True)
    slots_ref[...] = jnp.where(
        lane == 0, slot_a, jnp.where(lane == 1, slot_b, 0.0)).astype(jnp.int32)
    wts_ref[...] = jnp.where(lane == 0, w1, jnp.where(lane == 1, w2, 0.0))

    nt_i = nt.astype(jnp.int32)
    tb_i = tb.astype(jnp.int32)
    jj = lax.broadcasted_iota(jnp.int32, (128, E), 0)
    ee = lax.broadcasted_iota(jnp.int32, (128, E), 1)
    used = jnp.logical_and(jj >= tb_i, jj < tb_i + nt_i)
    te_row = jnp.sum(jnp.where(used, ee + 1, 0), axis=-1, keepdims=True) - 1
    e_last = jnp.max(jnp.where(nt_i > 0, ee, 0), axis=-1, keepdims=True)
    e_last = jnp.max(e_last, axis=0, keepdims=True)
    te_row = jnp.where(te_row < 0, e_last, te_row)
    ntot = jnp.sum(nt_i, axis=-1, keepdims=True)
    te_full = jnp.where(jj == NT, ntot, te_row)
    te_ref[...] = te_full


def _router_call(hidden_states, gate_w):
    return pl.pallas_call(
        _router_body,
        in_specs=[
            pl.BlockSpec((T, D), lambda: (0, 0)),
            pl.BlockSpec((E, D), lambda: (0, 0)),
        ],
        out_specs=[
            pl.BlockSpec((T, E), lambda: (0, 0)),
            pl.BlockSpec((T, E), lambda: (0, 0)),
            pl.BlockSpec((128, E), lambda: (0, 0)),
        ],
        out_shape=[
            jax.ShapeDtypeStruct((T, E), jnp.int32),
            jax.ShapeDtypeStruct((T, E), jnp.float32),
            jax.ShapeDtypeStruct((128, E), jnp.int32),
        ],
    )(hidden_states, gate_w)


@functools.partial(
    pl.kernel,
    mesh=plsc.VectorSubcoreMesh(core_axis_name="c", subcore_axis_name="s"),
    out_type=jax.ShapeDtypeStruct((TOT, D), jnp.float32),
    scratch_types=[
        pltpu.VMEM((CH,), jnp.int32),
        pltpu.VMEM((CH,), jnp.int32),
        pltpu.VMEM((CH, D), jnp.float32),
        pltpu.SemaphoreType.DMA,
    ],
)
def _scatter_call(x_hbm, s0_hbm, s1_hbm, xs_hbm, i0_v, i1_v, rows_v, sem):
    wid = lax.axis_index("s") * 2 + lax.axis_index("c")
    start = wid * CH
    pltpu.sync_copy(s0_hbm.at[pl.ds(start, CH)], i0_v)
    pltpu.sync_copy(s1_hbm.at[pl.ds(start, CH)], i1_v)
    pltpu.sync_copy(x_hbm.at[pl.ds(start, CH)], rows_v)
    pltpu.async_copy(rows_v, xs_hbm.at[i0_v], sem).wait()
    pltpu.async_copy(rows_v, xs_hbm.at[i1_v], sem).wait()


def _mm_body(te_ref, xs_ref, wg_ref, wu_ref, wd_ref, ys_ref):
    j = pl.program_id(0)

    @pl.when(j < te_ref[NT])
    def _():
        x = xs_ref[...].astype(jnp.bfloat16)
        g = lax.dot_general(x, wg_ref[0], (((1,), (1,)), ((), ())),
                            preferred_element_type=jnp.float32)
        u = lax.dot_general(x, wu_ref[0], (((1,), (1,)), ((), ())),
                            preferred_element_type=jnp.float32)
        h = (g * jax.nn.sigmoid(g)) * u
        ys_ref[...] = lax.dot_general(
            h.astype(jnp.bfloat16), wd_ref[0], (((1,), (1,)), ((), ())),
            preferred_element_type=jnp.float32)


def _mm_call(te_arr, xs, wg16, wu16, wd16):
    grid_spec = pltpu.PrefetchScalarGridSpec(
        num_scalar_prefetch=1,
        grid=(NT,),
        in_specs=[
            pl.BlockSpec((BT, D), lambda j, te: (j, 0)),
            pl.BlockSpec((1, FF, D), lambda j, te: (te[j], 0, 0)),
            pl.BlockSpec((1, FF, D), lambda j, te: (te[j], 0, 0)),
            pl.BlockSpec((1, D, FF), lambda j, te: (te[j], 0, 0)),
        ],
        out_specs=pl.BlockSpec((BT, D), lambda j, te: (j, 0)),
    )
    return pl.pallas_call(
        _mm_body,
        grid_spec=grid_spec,
        out_shape=jax.ShapeDtypeStruct((TOT, D), jnp.float32),
    )(te_arr, xs, wg16, wu16, wd16)


@functools.partial(
    pl.kernel,
    mesh=plsc.VectorSubcoreMesh(core_axis_name="c", subcore_axis_name="s"),
    out_type=[
        jax.ShapeDtypeStruct((T, D), jnp.float32),
        jax.ShapeDtypeStruct((T, D), jnp.float32),
    ],
    scratch_types=[
        pltpu.VMEM((CH,), jnp.int32),
        pltpu.VMEM((CH,), jnp.int32),
        pltpu.VMEM((CH, D), jnp.float32),
        pltpu.VMEM((CH, D), jnp.float32),
        pltpu.SemaphoreType.DMA,
    ],
)
def _gather_call(ys_hbm, s0_hbm, s1_hbm, g0_hbm, g1_hbm,
                 i0_v, i1_v, y0_v, y1_v, sem):
    wid = lax.axis_index("s") * 2 + lax.axis_index("c")
    start = wid * CH
    pltpu.sync_copy(s0_hbm.at[pl.ds(start, CH)], i0_v)
    pltpu.sync_copy(s1_hbm.at[pl.ds(start, CH)], i1_v)
    pltpu.async_copy(ys_hbm.at[i0_v], y0_v, sem).wait()
    pltpu.async_copy(ys_hbm.at[i1_v], y1_v, sem).wait()
    pltpu.sync_copy(y0_v, g0_hbm.at[pl.ds(start, CH)])
    pltpu.sync_copy(y1_v, g1_hbm.at[pl.ds(start, CH)])


def _comb_body(w_ref, g0_ref, g1_ref, out_ref):
    lane = lax.broadcasted_iota(jnp.int32, (T, E), 1)
    w = w_ref[...]
    w1 = jnp.sum(jnp.where(lane == 0, w, 0.0), axis=-1, keepdims=True)
    w2 = jnp.sum(jnp.where(lane == 1, w, 0.0), axis=-1, keepdims=True)
    out_ref[...] = w1 * g0_ref[...] + w2 * g1_ref[...]


def _comb_call(wts, g0, g1):
    return pl.pallas_call(
        _comb_body,
        in_specs=[
            pl.BlockSpec((T, E), lambda: (0, 0)),
            pl.BlockSpec((T, D), lambda: (0, 0)),
            pl.BlockSpec((T, D), lambda: (0, 0)),
        ],
        out_specs=pl.BlockSpec((T, D), lambda: (0, 0)),
        out_shape=jax.ShapeDtypeStruct((T, D), jnp.float32),
    )(wts, g0, g1)


def kernel(hidden_states, gate_w, w_gate, w_up, w_down):
    slots, wts, te_out = _router_call(hidden_states, gate_w)
    s0 = slots[:, 0]
    s1 = slots[:, 1]
    te_arr = te_out[:NT + 1, 0]
    xs = _scatter_call(hidden_states, s0, s1)
    ys = _mm_call(te_arr, xs, w_gate.astype(jnp.bfloat16),
                  w_up.astype(jnp.bfloat16), w_down.astype(jnp.bfloat16))
    g0, g1 = _gather_call(ys, s0, s1)
    return _comb_call(wts, g0, g1)


# speedup vs baseline: 1.0105x; 1.0105x over previous
import functools
import jax
import jax.numpy as jnp
from jax import lax
from jax.experimental import pallas as pl
from jax.experimental.pallas import tpu as pltpu
from jax.experimental.pallas import tpu_sc as plsc

T = 2048
D = 768
E = 8
FF = 768
BT = 256
NT = (2 * T) // BT + E
TOT = NT * BT
NW = 32
CH = T // NW


def _router_body(x_ref, gw_ref, slots_ref, wts_ref, te_ref):
    x = x_ref[...]
    logits = lax.dot_general(x, gw_ref[...], (((1,), (1,)), ((), ())),
                             preferred_element_type=jnp.float32)
    lane = lax.broadcasted_iota(jnp.int32, (T, E), 1)
    i1 = jnp.argmax(logits, axis=-1)[:, None]
    l1 = jnp.max(logits, axis=-1, keepdims=True)
    masked = jnp.where(lane == i1, -jnp.inf, logits)
    i2 = jnp.argmax(masked, axis=-1)[:, None]
    l2 = jnp.max(masked, axis=-1, keepdims=True)
    z = jnp.exp(l2 - l1)
    w1 = 1.0 / (1.0 + z)
    w2 = 1.0 - w1

    onehot = jnp.logical_or(lane == i1, lane == i2).astype(jnp.bfloat16)
    r_iota = lax.broadcasted_iota(jnp.int32, (T, T), 0)
    c_iota = lax.broadcasted_iota(jnp.int32, (T, T), 1)
    lt = (c_iota < r_iota).astype(jnp.bfloat16)
    ranks = lax.dot_general(lt, onehot, (((1,), (0,)), ((), ())),
                            preferred_element_type=jnp.float32)
    counts = jnp.sum(onehot.astype(jnp.float32), axis=0, keepdims=True)
    nt = jnp.floor((counts + (BT - 1)) / BT)
    ei = lax.broadcasted_iota(jnp.int32, (E, E), 0)
    ej = lax.broadcasted_iota(jnp.int32, (E, E), 1)
    m = (ei < ej).astype(jnp.float32)
    tb = lax.dot_general(nt, m, (((1,), (0,)), ((), ())),
                         preferred_element_type=jnp.float32)
    base = tb * BT

    slot_a = jnp.sum(jnp.where(lane == i1, ranks + base, 0.0), axis=-1,
                     keepdims=True)
    slot_b = jnp.sum(jnp.where(lane == i2, ranks + base, 0.0), axis=-1,
                     keepdims=True)
    slots_ref[...] = jnp.where(
        lane == 0, slot_a, jnp.where(lane == 1, slot_b, 0.0)).astype(jnp.int32)
    wts_ref[...] = jnp.where(lane == 0, w1, jnp.where(lane == 1, w2, 0.0))

    nt_i = nt.astype(jnp.int32)
    tb_i = tb.astype(jnp.int32)
    jj = lax.broadcasted_iota(jnp.int32, (128, E), 0)
    ee = lax.broadcasted_iota(jnp.int32, (128, E), 1)
    used = jnp.logical_and(jj >= tb_i, jj < tb_i + nt_i)
    te_row = jnp.sum(jnp.where(used, ee + 1, 0), axis=-1, keepdims=True) - 1
    e_last = jnp.max(jnp.where(nt_i > 0, ee, 0), axis=-1, keepdims=True)
    e_last = jnp.max(e_last, axis=0, keepdims=True)
    te_row = jnp.where(te_row < 0, e_last, te_row)
    ntot = jnp.sum(nt_i, axis=-1, keepdims=True)
    te_full = jnp.where(jj == NT, ntot, te_row)
    te_ref[...] = te_full


def _router_call(hidden_states, gate_w):
    return pl.pallas_call(
        _router_body,
        in_specs=[
            pl.BlockSpec((T, D), lambda: (0, 0)),
            pl.BlockSpec((E, D), lambda: (0, 0)),
        ],
        out_specs=[
            pl.BlockSpec((T, E), lambda: (0, 0)),
            pl.BlockSpec((T, E), lambda: (0, 0)),
            pl.BlockSpec((128, E), lambda: (0, 0)),
        ],
        out_shape=[
            jax.ShapeDtypeStruct((T, E), jnp.int32),
            jax.ShapeDtypeStruct((T, E), jnp.float32),
            jax.ShapeDtypeStruct((128, E), jnp.int32),
        ],
    )(hidden_states, gate_w)


@functools.partial(
    pl.kernel,
    mesh=plsc.VectorSubcoreMesh(core_axis_name="c", subcore_axis_name="s"),
    out_type=jax.ShapeDtypeStruct((TOT, D), jnp.float32),
    scratch_types=[
        pltpu.VMEM((CH,), jnp.int32),
        pltpu.VMEM((CH,), jnp.int32),
        pltpu.VMEM((CH, D), jnp.float32),
        pltpu.SemaphoreType.DMA,
        pltpu.SemaphoreType.DMA,
    ],
)
def _scatter_call(x_hbm, s0_hbm, s1_hbm, xs_hbm, i0_v, i1_v, rows_v,
                  sem0, sem1):
    wid = lax.axis_index("s") * 2 + lax.axis_index("c")
    start = wid * CH
    pltpu.sync_copy(s0_hbm.at[pl.ds(start, CH)], i0_v)
    pltpu.sync_copy(s1_hbm.at[pl.ds(start, CH)], i1_v)
    pltpu.sync_copy(x_hbm.at[pl.ds(start, CH)], rows_v)
    c0 = pltpu.async_copy(rows_v, xs_hbm.at[i0_v], sem0)
    c1 = pltpu.async_copy(rows_v, xs_hbm.at[i1_v], sem1)
    c0.wait()
    c1.wait()


def _mm_body(te_ref, xs_ref, wg_ref, wu_ref, wd_ref, ys_ref):
    j = pl.program_id(0)

    @pl.when(j < te_ref[NT])
    def _():
        x = xs_ref[...].astype(jnp.bfloat16)
        g = lax.dot_general(x, wg_ref[0], (((1,), (1,)), ((), ())),
                            preferred_element_type=jnp.float32)
        u = lax.dot_general(x, wu_ref[0], (((1,), (1,)), ((), ())),
                            preferred_element_type=jnp.float32)
        h = (g * jax.nn.sigmoid(g)) * u
        ys_ref[...] = lax.dot_general(
            h.astype(jnp.bfloat16), wd_ref[0], (((1,), (1,)), ((), ())),
            preferred_element_type=jnp.float32)


def _mm_call(te_arr, xs, wg16, wu16, wd16):
    grid_spec = pltpu.PrefetchScalarGridSpec(
        num_scalar_prefetch=1,
        grid=(NT,),
        in_specs=[
            pl.BlockSpec((BT, D), lambda j, te: (j, 0)),
            pl.BlockSpec((1, FF, D), lambda j, te: (te[j], 0, 0)),
            pl.BlockSpec((1, FF, D), lambda j, te: (te[j], 0, 0)),
            pl.BlockSpec((1, D, FF), lambda j, te: (te[j], 0, 0)),
        ],
        out_specs=pl.BlockSpec((BT, D), lambda j, te: (j, 0)),
    )
    return pl.pallas_call(
        _mm_body,
        grid_spec=grid_spec,
        out_shape=jax.ShapeDtypeStruct((TOT, D), jnp.float32),
    )(te_arr, xs, wg16, wu16, wd16)


@functools.partial(
    pl.kernel,
    mesh=plsc.VectorSubcoreMesh(core_axis_name="c", subcore_axis_name="s"),
    out_type=[
        jax.ShapeDtypeStruct((T, D), jnp.float32),
        jax.ShapeDtypeStruct((T, D), jnp.float32),
    ],
    scratch_types=[
        pltpu.VMEM((CH,), jnp.int32),
        pltpu.VMEM((CH,), jnp.int32),
        pltpu.VMEM((CH, D), jnp.float32),
        pltpu.VMEM((CH, D), jnp.float32),
        pltpu.SemaphoreType.DMA,
        pltpu.SemaphoreType.DMA,
        pltpu.SemaphoreType.DMA,
        pltpu.SemaphoreType.DMA,
    ],
)
def _gather_call(ys_hbm, s0_hbm, s1_hbm, g0_hbm, g1_hbm,
                 i0_v, i1_v, y0_v, y1_v, sem0, sem1, sem2, sem3):
    wid = lax.axis_index("s") * 2 + lax.axis_index("c")
    start = wid * CH
    pltpu.sync_copy(s0_hbm.at[pl.ds(start, CH)], i0_v)
    pltpu.sync_copy(s1_hbm.at[pl.ds(start, CH)], i1_v)
    c0 = pltpu.async_copy(ys_hbm.at[i0_v], y0_v, sem0)
    c1 = pltpu.async_copy(ys_hbm.at[i1_v], y1_v, sem1)
    c0.wait()
    s0c = pltpu.async_copy(y0_v, g0_hbm.at[pl.ds(start, CH)], sem2)
    c1.wait()
    s1c = pltpu.async_copy(y1_v, g1_hbm.at[pl.ds(start, CH)], sem3)
    s0c.wait()
    s1c.wait()


def _comb_body(w_ref, g0_ref, g1_ref, out_ref):
    lane = lax.broadcasted_iota(jnp.int32, (T, E), 1)
    w = w_ref[...]
    w1 = jnp.sum(jnp.where(lane == 0, w, 0.0), axis=-1, keepdims=True)
    w2 = jnp.sum(jnp.where(lane == 1, w, 0.0), axis=-1, keepdims=True)
    out_ref[...] = w1 * g0_ref[...] + w2 * g1_ref[...]


def _comb_call(wts, g0, g1):
    return pl.pallas_call(
        _comb_body,
        in_specs=[
            pl.BlockSpec((T, E), lambda: (0, 0)),
            pl.BlockSpec((T, D), lambda: (0, 0)),
            pl.BlockSpec((T, D), lambda: (0, 0)),
        ],
        out_specs=pl.BlockSpec((T, D), lambda: (0, 0)),
        out_shape=jax.ShapeDtypeStruct((T, D), jnp.float32),
    )(wts, g0, g1)


def kernel(hidden_states, gate_w, w_gate, w_up, w_down):
    slots, wts, te_out = _router_call(hidden_states, gate_w)
    s0 = slots[:, 0]
    s1 = slots[:, 1]
    te_arr = te_out[:NT + 1, 0]
    xs = _scatter_call(hidden_states, s0, s1)
    ys = _mm_call(te_arr, xs, w_gate.astype(jnp.bfloat16),
                  w_up.astype(jnp.bfloat16), w_down.astype(jnp.bfloat16))
    g0, g1 = _gather_call(ys, s0, s1)
    return _comb_call(wts, g0, g1)
